# Initial kernel scaffold; baseline (speedup 1.0000x reference)
import math

import jax
import jax.numpy as jnp
from jax.experimental import pallas as pl

N_NODES = 100000
CLUST_SIZE = 512
D_DATA = 5
D_NODE = 16
MAX_DIST = 5.0
K_NN = 3
H_EDGE = 32
H_GLOB = 64
N_CLASSES = 2

_HI = jax.lax.Precision.HIGHEST
_LOG_S = float(math.log(float(CLUST_SIZE)))


def _cluster_kernel(pts_ref, ptsT_ref, We1_ref, be1_ref, We2_ref, be2_ref,
                    Wroot_ref, broot_ref, Wg1_ref, bg1_ref, Wg2_ref, bg2_ref,
                    out_ref):
    S = CLUST_SIZE
    pts = pts_ref[...]
    xyz = pts[:, 0:3]
    val = pts[:, 4:5]

    cent = jnp.mean(xyz, axis=0, keepdims=True)
    rel = xyz - cent
    dist = jnp.sqrt(jnp.sum(rel * rel, axis=1, keepdims=True) + 1e-12)
    var = jnp.mean(jnp.square(xyz - cent), axis=0, keepdims=True)
    std = jnp.sqrt(var)
    dnorm = jnp.clip(dist / MAX_DIST, 0.0, 1.0)
    ones = jnp.ones_like(dist)
    stdb = jnp.broadcast_to(std, rel.shape)
    centb = jnp.broadcast_to(cent, rel.shape)
    x = jnp.concatenate([xyz, rel, dist, dnorm, val, stdb, centb, ones],
                        axis=1)

    ext = (jnp.max(xyz, axis=0, keepdims=True)
           - jnp.min(xyz, axis=0, keepdims=True))
    mval = jnp.mean(val, axis=0, keepdims=True)
    sval = jnp.sqrt(jnp.mean(jnp.square(val - mval), axis=0, keepdims=True))
    tval = jnp.sum(val, axis=0, keepdims=True)
    rms = jnp.sqrt(jnp.mean(dist * dist, axis=0, keepdims=True))
    dmax = jnp.max(dist, axis=0, keepdims=True)
    lsize = jnp.full((1, 1), _LOG_S, dtype=jnp.float32)
    one = jnp.ones((1, 1), dtype=jnp.float32)
    u = jnp.concatenate([cent, std, ext, mval, sval, tval, rms, dmax,
                         lsize, one], axis=1)

    d2 = None
    for c in range(3):
        col = pts[:, c:c + 1]
        row = ptsT_ref[c:c + 1, :]
        diff = col - row
        sq = diff * diff
        d2 = sq if d2 is None else d2 + sq
    ii = jax.lax.broadcasted_iota(jnp.float32, (S, S), 0)
    jj = jax.lax.broadcasted_iota(jnp.float32, (S, S), 1)
    d2 = d2 + jnp.where(ii == jj, 1e10, 0.0)

    x_src = []
    for _ in range(K_NN):
        m = jnp.min(d2, axis=1, keepdims=True)
        amin = jnp.min(jnp.where(d2 == m, jj, 1e9), axis=1,
                       keepdims=True)
        onehot = (jj == amin).astype(jnp.float32)
        g = jnp.dot(onehot, x, precision=_HI)
        x_src.append(g)
        d2 = jnp.where(jj == amin, 1e10, d2)

    agg = jnp.zeros((S, D_NODE), dtype=jnp.float32)
    for k in range(K_NN):
        g = x_src[k]
        disp = g[:, 0:3] - xyz
        edist = jnp.sqrt(jnp.sum(disp * disp, axis=1, keepdims=True) + 1e-12)
        e = jnp.concatenate([disp, edist], axis=1)
        h_e = jax.nn.relu(jnp.dot(e, We1_ref[...], precision=_HI)
                          + be1_ref[...])
        W_e = jnp.dot(h_e, We2_ref[...], precision=_HI) + be2_ref[...]
        msg = jnp.zeros((S, D_NODE), dtype=jnp.float32)
        for d in range(D_NODE):
            msg = msg + g[:, d:d + 1] * W_e[:, d * D_NODE:(d + 1) * D_NODE]
        agg = agg + msg

    h = jax.nn.relu(jnp.dot(x, Wroot_ref[...], precision=_HI)
                    + broot_ref[...] + agg)
    pooled = jnp.sum(h, axis=0, keepdims=True) / float(S)
    g_in = jnp.concatenate([pooled, u], axis=1)
    hg = jax.nn.relu(jnp.dot(g_in, Wg1_ref[...], precision=_HI)
                     + bg1_ref[...])
    out_ref[...] = (jnp.dot(hg, Wg2_ref[...], precision=_HI)
                    + bg2_ref[...])


@jax.jit
def kernel(data, clusts, We1, be1, We2, be2, Wroot, broot, Wg1, bg1, Wg2,
           bg2):
    idx = clusts[0]
    pts = jnp.take(data, idx, axis=0)
    ptsT = pts.T
    out = pl.pallas_call(
        _cluster_kernel,
        out_shape=jax.ShapeDtypeStruct((1, N_CLASSES), jnp.float32),
    )(pts, ptsT, We1, be1.reshape(1, -1), We2, be2.reshape(1, -1),
      Wroot, broot.reshape(1, -1), Wg1, bg1.reshape(1, -1), Wg2,
      bg2.reshape(1, -1))
    return out.reshape(N_CLASSES)


# speedup vs baseline: 67.6781x; 67.6781x over previous
import math

import jax
import jax.numpy as jnp
from jax.experimental import pallas as pl

N_NODES = 100000
CLUST_SIZE = 512
D_DATA = 5
D_NODE = 16
MAX_DIST = 5.0
K_NN = 3
H_EDGE = 32
H_GLOB = 64
N_CLASSES = 2

_HI = jax.lax.Precision.HIGHEST
_LOG_S = float(math.log(float(CLUST_SIZE)))


def _cluster_kernel(pts_ref, ptsT_ref, We1_ref, be1_ref, We2_ref, be2_ref,
                    Wroot_ref, broot_ref, Wg1_ref, bg1_ref, Wg2_ref, bg2_ref,
                    out_ref):
    S = CLUST_SIZE
    pts = pts_ref[...]
    xyz = pts[:, 0:3]
    val = pts[:, 4:5]

    cent = jnp.mean(xyz, axis=0, keepdims=True)
    rel = xyz - cent
    dist = jnp.sqrt(jnp.sum(rel * rel, axis=1, keepdims=True) + 1e-12)
    var = jnp.mean(jnp.square(xyz - cent), axis=0, keepdims=True)
    std = jnp.sqrt(var)
    dnorm = jnp.clip(dist / MAX_DIST, 0.0, 1.0)
    ones = jnp.ones_like(dist)
    stdb = jnp.broadcast_to(std, rel.shape)
    centb = jnp.broadcast_to(cent, rel.shape)
    x = jnp.concatenate([xyz, rel, dist, dnorm, val, stdb, centb, ones],
                        axis=1)

    ext = (jnp.max(xyz, axis=0, keepdims=True)
           - jnp.min(xyz, axis=0, keepdims=True))
    mval = jnp.mean(val, axis=0, keepdims=True)
    sval = jnp.sqrt(jnp.mean(jnp.square(val - mval), axis=0, keepdims=True))
    tval = jnp.sum(val, axis=0, keepdims=True)
    rms = jnp.sqrt(jnp.mean(dist * dist, axis=0, keepdims=True))
    dmax = jnp.max(dist, axis=0, keepdims=True)
    lsize = jnp.full((1, 1), _LOG_S, dtype=jnp.float32)
    one = jnp.ones((1, 1), dtype=jnp.float32)
    u = jnp.concatenate([cent, std, ext, mval, sval, tval, rms, dmax,
                         lsize, one], axis=1)

    d2 = None
    for c in range(3):
        col = pts[:, c:c + 1]
        row = ptsT_ref[c:c + 1, :]
        diff = col - row
        sq = diff * diff
        d2 = sq if d2 is None else d2 + sq
    ii = jax.lax.broadcasted_iota(jnp.int32, (S, S), 0).astype(jnp.float32)
    jj = jax.lax.broadcasted_iota(jnp.int32, (S, S), 1).astype(jnp.float32)
    d2 = d2 + jnp.where(ii == jj, 1e10, 0.0)

    x_src = []
    for _ in range(K_NN):
        m = jnp.min(d2, axis=1, keepdims=True)
        amin = jnp.min(jnp.where(d2 == m, jj, 1e9), axis=1,
                       keepdims=True)
        onehot = (jj == amin).astype(jnp.float32)
        g = jnp.dot(onehot, x, precision=_HI)
        x_src.append(g)
        d2 = jnp.where(jj == amin, 1e10, d2)

    agg = jnp.zeros((S, D_NODE), dtype=jnp.float32)
    for k in range(K_NN):
        g = x_src[k]
        disp = g[:, 0:3] - xyz
        edist = jnp.sqrt(jnp.sum(disp * disp, axis=1, keepdims=True) + 1e-12)
        e = jnp.concatenate([disp, edist], axis=1)
        h_e = jax.nn.relu(jnp.dot(e, We1_ref[...], precision=_HI)
                          + be1_ref[...])
        W_e = jnp.dot(h_e, We2_ref[...], precision=_HI) + be2_ref[...]
        msg = jnp.zeros((S, D_NODE), dtype=jnp.float32)
        for d in range(D_NODE):
            msg = msg + g[:, d:d + 1] * W_e[:, d * D_NODE:(d + 1) * D_NODE]
        agg = agg + msg

    h = jax.nn.relu(jnp.dot(x, Wroot_ref[...], precision=_HI)
                    + broot_ref[...] + agg)
    pooled = jnp.sum(h, axis=0, keepdims=True) / float(S)
    g_in = jnp.concatenate([pooled, u], axis=1)
    hg = jax.nn.relu(jnp.dot(g_in, Wg1_ref[...], precision=_HI)
                     + bg1_ref[...])
    out_ref[...] = (jnp.dot(hg, Wg2_ref[...], precision=_HI)
                    + bg2_ref[...])


@jax.jit
def kernel(data, clusts, We1, be1, We2, be2, Wroot, broot, Wg1, bg1, Wg2,
           bg2):
    idx = clusts[0]
    pts = jnp.take(data, idx, axis=0)
    ptsT = pts.T
    out = pl.pallas_call(
        _cluster_kernel,
        out_shape=jax.ShapeDtypeStruct((1, N_CLASSES), jnp.float32),
    )(pts, ptsT, We1, be1.reshape(1, -1), We2, be2.reshape(1, -1),
      Wroot, broot.reshape(1, -1), Wg1, bg1.reshape(1, -1), Wg2,
      bg2.reshape(1, -1))
    return out.reshape(N_CLASSES)
